# baseline (speedup 1.0000x reference)
import jax
import jax.numpy as jnp
from jax import lax
from jax.experimental import pallas as pl
from jax.experimental.pallas import tpu as pltpu
from jax.experimental.pallas import tpu_sc as plsc

_B = 16384
_D = 32
_RPL = 128 // _D
_NC = 2
_NS = 16
_NW = _NC * _NS
_BPW = _B // _NW
_NPIECE = 4
_PIECE = _BPW // _NPIECE
_NCHUNK = _PIECE // 16
_NRV = 1000
_RV_LINES = _NRV * _D // 128


def _sc_score(E_hbm, rv_hbm, u_hbm, r_hbm, v_hbm, out_hbm,
              u_idx_v, r_idx_v, v_idx_v, u_line_v, v_line_v,
              rv_l, u_l0, u_l1, v_l0, v_l1, out_v,
              sem_rv, sem_u0, sem_u1, sem_v0, sem_v1):
    wid = lax.axis_index("s") * _NC + lax.axis_index("c")
    base = wid * _BPW

    crv = pltpu.async_copy(rv_hbm, rv_l, sem_rv)

    pltpu.sync_copy(u_hbm.at[pl.ds(base, _BPW)], u_idx_v)
    pltpu.sync_copy(v_hbm.at[pl.ds(base, _BPW)], v_idx_v)
    pltpu.sync_copy(r_hbm.at[pl.ds(base, _BPW)], r_idx_v)

    for s in range(_BPW // 16):
        sl = pl.ds(s * 16, 16)
        u_line_v[sl] = lax.shift_right_logical(u_idx_v[sl], 2)
        v_line_v[sl] = lax.shift_right_logical(v_idx_v[sl], 2)

    u_bufs = (u_l0, u_l1)
    v_bufs = (v_l0, v_l1)
    u_sems = (sem_u0, sem_u1)
    v_sems = (sem_v0, sem_v1)

    def fire(p):
        sl = pl.ds(p * _PIECE, _PIECE)
        cu = pltpu.async_copy(E_hbm.at[u_line_v.at[sl]], u_bufs[p % 2],
                              u_sems[p % 2])
        cv = pltpu.async_copy(E_hbm.at[v_line_v.at[sl]], v_bufs[p % 2],
                              v_sems[p % 2])
        return cu, cv

    lanes = lax.iota(jnp.int32, 16)
    three = jnp.full((16,), 3, jnp.int32)

    pend = fire(0)
    crv.wait()

    for p in range(_NPIECE):
        pend[0].wait()
        pend[1].wait()
        if p + 1 < _NPIECE:
            pend = fire(p + 1)
        u_buf = u_bufs[p % 2]
        v_buf = v_bufs[p % 2]

        def chunk_body(c, carry, p=p, u_buf=u_buf, v_buf=v_buf):
            b0 = p * _PIECE + c * 16
            rowloc = c * 16 + lanes
            uidx = u_idx_v[pl.ds(b0, 16)]
            vidx = v_idx_v[pl.ds(b0, 16)]
            ridx = r_idx_v[pl.ds(b0, 16)]
            usub = lax.shift_left(jnp.bitwise_and(uidx, three), 5)
            vsub = lax.shift_left(jnp.bitwise_and(vidx, three), 5)
            rline = lax.shift_right_logical(ridx, 2)
            rsub = lax.shift_left(jnp.bitwise_and(ridx, three), 5)
            acc = jnp.zeros((16,), jnp.float32)
            for d in range(_D):
                ud = plsc.load_gather(u_buf, [rowloc, usub + d])
                vd = plsc.load_gather(v_buf, [rowloc, vsub + d])
                rd = plsc.load_gather(rv_l, [rline, rsub + d])
                t = ud - (vd + rd)
                acc = acc + t * t
            out_v[pl.ds(b0, 16)] = -acc
            return carry

        lax.fori_loop(0, _NCHUNK, chunk_body, 0)

    pltpu.sync_copy(out_v, out_hbm.at[pl.ds(base, _BPW)])


@jax.jit
def kernel(E, rv, u_idx, r_idx, v_idx):
    E_lines = E.reshape(-1, 128)
    rv_lines = rv.reshape(-1, 128)
    mesh = plsc.VectorSubcoreMesh(core_axis_name="c", subcore_axis_name="s")
    run = pl.kernel(
        _sc_score,
        out_type=jax.ShapeDtypeStruct((_B,), jnp.float32),
        mesh=mesh,
        compiler_params=pltpu.CompilerParams(needs_layout_passes=False),
        scratch_types=[
            pltpu.VMEM((_BPW,), jnp.int32),
            pltpu.VMEM((_BPW,), jnp.int32),
            pltpu.VMEM((_BPW,), jnp.int32),
            pltpu.VMEM((_BPW,), jnp.int32),
            pltpu.VMEM((_BPW,), jnp.int32),
            pltpu.VMEM((_RV_LINES, 128), jnp.float32),
            pltpu.VMEM((_PIECE, 128), jnp.float32),
            pltpu.VMEM((_PIECE, 128), jnp.float32),
            pltpu.VMEM((_PIECE, 128), jnp.float32),
            pltpu.VMEM((_PIECE, 128), jnp.float32),
            pltpu.VMEM((_BPW,), jnp.float32),
            pltpu.SemaphoreType.DMA,
            pltpu.SemaphoreType.DMA,
            pltpu.SemaphoreType.DMA,
            pltpu.SemaphoreType.DMA,
            pltpu.SemaphoreType.DMA,
        ],
    )
    return run(E_lines, rv_lines, u_idx, r_idx, v_idx)
